# baseline (speedup 1.0000x reference)
import functools

import jax
import jax.numpy as jnp
from jax import lax
from jax.experimental import pallas as pl
from jax.experimental.pallas import tpu as pltpu
from jax.experimental.pallas import tpu_sc as plsc

N = 1000
E = 8000
D = 128
DE = 16
H = 128
T = 3

EP = 8192
CH = 128
NTAB = 1024
GTAB = 8192
KTAB = 1 << 20
NSUB = 16
EPT = EP // NSUB
NCHUNK = EPT // CH
NC2 = EP // (32 * CH)

_mesh = plsc.VectorSubcoreMesh(core_axis_name="c", subcore_axis_name="s")


def _zero_rows(buf):
    z = jnp.zeros((16,), jnp.float32)
    for r in range(CH):
        for c in range(H // 16):
            buf[r, pl.ds(c * 16, 16)] = z


@functools.partial(
    pl.kernel,
    out_type=(jax.ShapeDtypeStruct((2, EP), jnp.int32),
              jax.ShapeDtypeStruct((2, EP), jnp.int32),
              jax.ShapeDtypeStruct((EP, H), jnp.float32)),
    mesh=_mesh,
    scratch_types=[
        pltpu.VMEM((8192,), jnp.int32),
        pltpu.VMEM((NCHUNK, CH), jnp.int32),
        pltpu.VMEM((NCHUNK, CH), jnp.int32),
        pltpu.VMEM((NCHUNK, CH), jnp.int32),
        pltpu.VMEM((NCHUNK, CH), jnp.int32),
        pltpu.VMEM((2, CH, H), jnp.float32),
        pltpu.VMEM_SHARED((KTAB,), jnp.int32),
        pltpu.SemaphoreType.DMA,
        pltpu.SemaphoreType.DMA,
        pltpu.SemaphoreType.DMA,
    ],
)
def _sc_prep(xw_hbm, eb_hbm, srcp_hbm, dstp_hbm, sidx_hbm, gidx_hbm, h0_hbm,
             fill_v, s_v, d_v, k_v, r_v, hbuf, table, sem_a, sem_b, sem_z):
    cid = lax.axis_index("c")
    sid = lax.axis_index("s")
    base = sid * EPT

    @pl.when(cid == 0)
    def _():
        ins = [pltpu.async_copy(srcp_hbm.at[pl.ds(base + c * CH, CH)],
                                s_v.at[c], sem_z) for c in range(NCHUNK)]
        pltpu.sync_copy(dstp_hbm.at[pl.ds(base, EPT)], fill_v.at[pl.ds(0, EPT)])
        pltpu.sync_copy(fill_v.at[pl.ds(0, EPT)], sidx_hbm.at[0, pl.ds(base, EPT)])
        for dsc in ins:
            dsc.wait()
        sems = (sem_a, sem_b)
        out = [None, None]
        for c in range(NCHUNK):
            sl = c % 2
            if out[sl] is not None:
                out[sl].wait()
            pltpu.async_copy(eb_hbm.at[pl.ds(base + c * CH, CH)], hbuf.at[sl],
                             sems[sl]).wait()
            pltpu.async_copy(xw_hbm.at[s_v.at[c]], hbuf.at[sl], sems[sl],
                             add=True).wait()

            def relu_row(r, _):
                for j in range(H // 16):
                    cs = pl.ds(j * 16, 16)
                    hbuf[sl, r, cs] = jnp.maximum(hbuf[sl, r, cs], 0.0)
                return 0

            lax.fori_loop(0, CH, relu_row, 0)
            out[sl] = pltpu.async_copy(hbuf.at[sl],
                                       h0_hbm.at[pl.ds(base + c * CH, CH)],
                                       sems[sl])
        pltpu.sync_copy(srcp_hbm.at[pl.ds(base, EPT)], fill_v.at[pl.ds(0, EPT)])
        pltpu.sync_copy(fill_v.at[pl.ds(0, EPT)], gidx_hbm.at[0, pl.ds(base, EPT)])
        for d2 in out:
            if d2 is not None:
                d2.wait()

    @pl.when(cid == 1)
    def _():
        ins = []
        for c in range(NCHUNK):
            ins.append(pltpu.async_copy(srcp_hbm.at[pl.ds(base + c * CH, CH)],
                                        s_v.at[c], sem_a))
            ins.append(pltpu.async_copy(dstp_hbm.at[pl.ds(base + c * CH, CH)],
                                        d_v.at[c], sem_a))
        neg16 = jnp.full((16,), -1, jnp.int32)
        for t in range(8192 // 16):
            fill_v[pl.ds(t * 16, 16)] = neg16
        zs = [pltpu.async_copy(
                  fill_v, table.at[pl.ds(sid * (KTAB // NSUB) + z * 8192, 8192)],
                  sem_b) for z in range(KTAB // NSUB // 8192)]
        for dsc in ins:
            dsc.wait()
        for c in range(NCHUNK):
            for j in range(CH // 16):
                sl = pl.ds(j * 16, 16)
                k_v[c, sl] = s_v[c, sl] * N + d_v[c, sl]
                ids = lax.iota(jnp.int32, 16) + (base + c * CH + j * 16)
                r_v[c, sl] = jnp.minimum(ids, GTAB - 2)
        for dsc in zs:
            dsc.wait()
        plsc.subcore_barrier()
        sc = [pltpu.async_copy(r_v.at[c], table.at[k_v.at[c]], sem_a)
              for c in range(NCHUNK)]
        for dsc in sc:
            dsc.wait()
        plsc.subcore_barrier()
        ga = [pltpu.async_copy(table.at[k_v.at[c]], r_v.at[c], sem_a)
              for c in range(NCHUNK)]
        for dsc in ga:
            dsc.wait()
        outs = [pltpu.async_copy(r_v.at[c], sidx_hbm.at[1, pl.ds(base + c * CH, CH)],
                                 sem_b) for c in range(NCHUNK)]
        for c in range(NCHUNK):
            for j in range(CH // 16):
                sl = pl.ds(j * 16, 16)
                k_v[c, sl] = d_v[c, sl] * N + s_v[c, sl]
        for dsc in outs:
            dsc.wait()
        gb = [pltpu.async_copy(table.at[k_v.at[c]], r_v.at[c], sem_a)
              for c in range(NCHUNK)]
        for dsc in gb:
            dsc.wait()
        for c in range(NCHUNK):
            for j in range(CH // 16):
                sl = pl.ds(j * 16, 16)
                rv = r_v[c, sl]
                r_v[c, sl] = jnp.where(rv < 0, GTAB - 1, rv)
        outs = [pltpu.async_copy(r_v.at[c], gidx_hbm.at[1, pl.ds(base + c * CH, CH)],
                                 sem_b) for c in range(NCHUNK)]
        for dsc in outs:
            dsc.wait()


@functools.partial(
    pl.kernel,
    out_type=jax.ShapeDtypeStruct((2, EP, H), jnp.float32),
    mesh=_mesh,
    scratch_types=[
        pltpu.VMEM((NCHUNK, CH), jnp.int32),
        pltpu.VMEM((NCHUNK, CH), jnp.int32),
        pltpu.VMEM((2, CH, H), jnp.float32),
        pltpu.VMEM((CH, H), jnp.float32),
        pltpu.VMEM_SHARED((GTAB, H), jnp.float32),
        pltpu.SemaphoreType.DMA,
        pltpu.SemaphoreType.DMA,
        pltpu.SemaphoreType.DMA,
    ],
)
def _sc_round(h_hbm, sidx_hbm, gidx_hbm, m_hbm, siv, giv, hbuf, zbuf, table,
              sem_a, sem_b, sem_z):
    cid = lax.axis_index("c")
    sid = lax.axis_index("s")
    base = sid * EPT
    sems = (sem_a, sem_b)
    ins = []
    for c in range(NCHUNK):
        ins.append(pltpu.async_copy(sidx_hbm.at[cid, pl.ds(base + c * CH, CH)],
                                    siv.at[c], sem_z))
        ins.append(pltpu.async_copy(gidx_hbm.at[cid, pl.ds(base + c * CH, CH)],
                                    giv.at[c], sem_z))
    ld = [pltpu.async_copy(h_hbm.at[pl.ds(base + c * CH, CH)], hbuf.at[c],
                           sems[c]) for c in range(2)]
    _zero_rows(zbuf)
    zs = [pltpu.async_copy(zbuf, table.at[pl.ds(sid * (GTAB // NSUB) + z * CH, CH)],
                           sem_z) for z in range(GTAB // NSUB // CH)]
    for dsc in ins:
        dsc.wait()
    for dsc in zs:
        dsc.wait()
    ld2 = pltpu.async_copy(h_hbm.at[pl.ds(base + 2 * CH, CH)], zbuf, sem_z)
    for dsc in ld:
        dsc.wait()
    plsc.subcore_barrier()
    sc0 = pltpu.async_copy(hbuf.at[0], table.at[siv.at[0]], sem_a, add=True)
    sc1 = pltpu.async_copy(hbuf.at[1], table.at[siv.at[1]], sem_b, add=True)
    ld2.wait()
    sc2 = pltpu.async_copy(zbuf, table.at[siv.at[2]], sem_z, add=True)
    sc0.wait()
    pltpu.async_copy(h_hbm.at[pl.ds(base + 3 * CH, CH)], hbuf.at[0], sem_a).wait()
    sc3 = pltpu.async_copy(hbuf.at[0], table.at[siv.at[3]], sem_a, add=True)
    sc1.wait()
    sc2.wait()
    sc3.wait()
    plsc.subcore_barrier()
    g0 = pltpu.async_copy(table.at[giv.at[0]], hbuf.at[0], sem_a)
    g1 = pltpu.async_copy(table.at[giv.at[1]], hbuf.at[1], sem_b)
    g2 = pltpu.async_copy(table.at[giv.at[2]], zbuf, sem_z)
    g0.wait()
    o0 = pltpu.async_copy(hbuf.at[0], m_hbm.at[cid, pl.ds(base, CH)], sem_a)
    g1.wait()
    o1 = pltpu.async_copy(hbuf.at[1], m_hbm.at[cid, pl.ds(base + CH, CH)], sem_b)
    g2.wait()
    o2 = pltpu.async_copy(zbuf, m_hbm.at[cid, pl.ds(base + 2 * CH, CH)], sem_z)
    o0.wait()
    pltpu.async_copy(table.at[giv.at[3]], hbuf.at[0], sem_a).wait()
    o3 = pltpu.async_copy(hbuf.at[0], m_hbm.at[cid, pl.ds(base + 3 * CH, CH)],
                          sem_a)
    o1.wait()
    o2.wait()
    o3.wait()


@functools.partial(
    pl.kernel,
    out_type=jax.ShapeDtypeStruct((NTAB, H), jnp.float32),
    mesh=_mesh,
    scratch_types=[
        pltpu.VMEM((NCHUNK, CH), jnp.int32),
        pltpu.VMEM((2, CH, H), jnp.float32),
        pltpu.VMEM((CH, H), jnp.float32),
        pltpu.VMEM_SHARED((NTAB, H), jnp.float32),
        pltpu.SemaphoreType.DMA,
        pltpu.SemaphoreType.DMA,
        pltpu.SemaphoreType.DMA,
    ],
)
def _sc_final(hw_hbm, dstp_hbm, xa_hbm, out_hbm, iv, hbuf, zbuf, table,
              sem_a, sem_b, sem_z):
    cid = lax.axis_index("c")
    sid = lax.axis_index("s")
    base = sid * EPT
    sems = (sem_a, sem_b)
    ins = [pltpu.async_copy(dstp_hbm.at[pl.ds(base + c * CH, CH)], iv.at[c],
                            sem_z) for c in range(NCHUNK)]
    ld = [pltpu.async_copy(hw_hbm.at[pl.ds(base + c * CH, CH)], hbuf.at[c],
                           sems[c]) for c in range(2)]
    _zero_rows(zbuf)
    rpt = NTAB // NSUB
    z = pltpu.async_copy(zbuf.at[pl.ds(0, rpt)], table.at[pl.ds(sid * rpt, rpt)],
                         sem_z)
    for dsc in ins:
        dsc.wait()
    z.wait()
    ld2 = pltpu.async_copy(hw_hbm.at[pl.ds(base + 2 * CH, CH)], zbuf, sem_z)
    for dsc in ld:
        dsc.wait()
    plsc.subcore_barrier()
    sc0 = pltpu.async_copy(hbuf.at[0], table.at[iv.at[0]], sem_a, add=True)
    sc1 = pltpu.async_copy(hbuf.at[1], table.at[iv.at[1]], sem_b, add=True)
    ld2.wait()
    sc2 = pltpu.async_copy(zbuf, table.at[iv.at[2]], sem_z, add=True)
    sc0.wait()
    pltpu.async_copy(hw_hbm.at[pl.ds(base + 3 * CH, CH)], hbuf.at[0], sem_a).wait()
    sc3 = pltpu.async_copy(hbuf.at[0], table.at[iv.at[3]], sem_a, add=True)
    sc1.wait()
    sc2.wait()
    sc3.wait()
    plsc.subcore_barrier()
    rpo = 512 // NSUB
    row0 = cid * 512 + sid * rpo
    pltpu.sync_copy(xa_hbm.at[pl.ds(row0, rpo)], hbuf.at[0, pl.ds(0, rpo)])
    pltpu.sync_copy(table.at[pl.ds(row0, rpo)], hbuf.at[1, pl.ds(0, rpo)])

    def orow(r, _):
        for c in range(H // 16):
            sl = pl.ds(c * 16, 16)
            hbuf[0, r, sl] = jnp.maximum(hbuf[0, r, sl] + hbuf[1, r, sl], 0.0)
        return 0

    lax.fori_loop(0, rpo, orow, 0)
    pltpu.sync_copy(hbuf.at[0, pl.ds(0, rpo)], out_hbm.at[pl.ds(row0, rpo)])


def _eb_body(ea_ref, w2_ref, b_ref, o_ref):
    o_ref[...] = (jnp.dot(ea_ref[...], w2_ref[...],
                          preferred_element_type=jnp.float32) + b_ref[...])


def _tc_eb(ea, w2t, b):
    return pl.pallas_call(
        _eb_body,
        grid=(EP // 1024,),
        in_specs=[
            pl.BlockSpec((1024, DE), lambda i: (i, 0)),
            pl.BlockSpec((DE, H), lambda i: (0, 0)),
            pl.BlockSpec((1, H), lambda i: (0, 0)),
        ],
        out_specs=pl.BlockSpec((1024, H), lambda i: (i, 0)),
        out_shape=jax.ShapeDtypeStruct((EP, H), jnp.float32),
    )(ea, w2t, b)


def _round_body(m_ref, h0_ref, w_ref, b_ref, o_ref):
    mm = m_ref[0] - m_ref[1]
    acc = (h0_ref[...]
           + jnp.dot(mm, w_ref[...], preferred_element_type=jnp.float32)
           + b_ref[...])
    o_ref[...] = jnp.maximum(acc, 0.0)


def _tc_round(m, h0, wt, b):
    return pl.pallas_call(
        _round_body,
        grid=(EP // 1024,),
        in_specs=[
            pl.BlockSpec((2, 1024, H), lambda i: (0, i, 0)),
            pl.BlockSpec((1024, H), lambda i: (i, 0)),
            pl.BlockSpec((H, H), lambda i: (0, 0)),
            pl.BlockSpec((1, H), lambda i: (0, 0)),
        ],
        out_specs=pl.BlockSpec((1024, H), lambda i: (i, 0)),
        out_shape=jax.ShapeDtypeStruct((EP, H), jnp.float32),
    )(m, h0, wt, b)


def _round_final_body(m_ref, h0_ref, w_ref, b_ref, w2_ref, o_ref):
    mm = m_ref[0] - m_ref[1]
    h3 = jnp.maximum(
        h0_ref[...]
        + jnp.dot(mm, w_ref[...], preferred_element_type=jnp.float32)
        + b_ref[...], 0.0)
    o_ref[...] = jnp.dot(h3, w2_ref[...], preferred_element_type=jnp.float32)


def _tc_round_final(m, h0, wt, b, wn2t):
    return pl.pallas_call(
        _round_final_body,
        grid=(EP // 1024,),
        in_specs=[
            pl.BlockSpec((2, 1024, H), lambda i: (0, i, 0)),
            pl.BlockSpec((1024, H), lambda i: (i, 0)),
            pl.BlockSpec((H, H), lambda i: (0, 0)),
            pl.BlockSpec((1, H), lambda i: (0, 0)),
            pl.BlockSpec((H, H), lambda i: (0, 0)),
        ],
        out_specs=pl.BlockSpec((1024, H), lambda i: (i, 0)),
        out_shape=jax.ShapeDtypeStruct((EP, H), jnp.float32),
    )(m, h0, wt, b, wn2t)


def _pre_body(x_ref, w1_ref, wn_ref, b_ref, xw_ref, xa_ref):
    xw_ref[...] = jnp.dot(x_ref[...], w1_ref[...],
                          preferred_element_type=jnp.float32)
    xa_ref[...] = (jnp.dot(x_ref[...], wn_ref[...],
                           preferred_element_type=jnp.float32) + b_ref[...])


def _tc_pre(xp, w1t, wn1t, b):
    return pl.pallas_call(
        _pre_body,
        grid=(NTAB // 512,),
        in_specs=[
            pl.BlockSpec((512, D), lambda i: (i, 0)),
            pl.BlockSpec((D, H), lambda i: (0, 0)),
            pl.BlockSpec((D, H), lambda i: (0, 0)),
            pl.BlockSpec((1, H), lambda i: (0, 0)),
        ],
        out_specs=[
            pl.BlockSpec((512, H), lambda i: (i, 0)),
            pl.BlockSpec((512, H), lambda i: (i, 0)),
        ],
        out_shape=[
            jax.ShapeDtypeStruct((NTAB, H), jnp.float32),
            jax.ShapeDtypeStruct((NTAB, H), jnp.float32),
        ],
    )(xp, w1t, wn1t, b)


def kernel(x, edge_index, edge_attr, W_edge_init, b_edge_init, W_msg, b_msg,
           W_node, b_node):
    src = edge_index[0]
    dst = edge_index[1]

    pad = EP - E
    srcp = jnp.concatenate([src, jnp.full((pad,), N, jnp.int32)])
    dstp = jnp.concatenate([dst, jnp.full((pad,), N, jnp.int32)])
    eap = jnp.pad(edge_attr, ((0, pad), (0, 0)))

    w1t = W_edge_init[:, :D].T
    w2t = W_edge_init[:, D:].T
    wmt = W_msg.T
    wn1t = W_node[:, :D].T
    wn2t = W_node[:, D:].T
    be = b_edge_init.reshape(1, H)
    bm = b_msg.reshape(1, H)
    bn = b_node.reshape(1, H)

    xp = jnp.pad(x, ((0, NTAB - N), (0, 0)))
    xw, xa = _tc_pre(xp, w1t, wn1t, bn)
    eb = _tc_eb(eap, w2t, be)
    sidx, gidx, h0 = _sc_prep(xw, eb, srcp, dstp)
    h = h0
    for t in range(T - 1):
        m = _sc_round(h, sidx, gidx)
        h = _tc_round(m, h0, wmt, bm)
    m = _sc_round(h, sidx, gidx)
    hw = _tc_round_final(m, h0, wmt, bm, wn2t)
    out = _sc_final(hw, dstp, xa)
    return out[:N]


# speedup vs baseline: 1.0067x; 1.0067x over previous
import functools

import jax
import jax.numpy as jnp
from jax import lax
from jax.experimental import pallas as pl
from jax.experimental.pallas import tpu as pltpu
from jax.experimental.pallas import tpu_sc as plsc

N = 1000
E = 8000
D = 128
DE = 16
H = 128
T = 3

EP = 8192
CH = 128
NTAB = 1024
GTAB = 8192
KTAB = 1 << 20
NSUB = 16
EPT = EP // NSUB
NCHUNK = EPT // CH
NC2 = EP // (32 * CH)

_mesh = plsc.VectorSubcoreMesh(core_axis_name="c", subcore_axis_name="s")


def _zero_rows(buf):
    z = jnp.zeros((16,), jnp.float32)
    for r in range(CH):
        for c in range(H // 16):
            buf[r, pl.ds(c * 16, 16)] = z


@functools.partial(
    pl.kernel,
    out_type=(jax.ShapeDtypeStruct((2, EP), jnp.int32),
              jax.ShapeDtypeStruct((2, EP), jnp.int32),
              jax.ShapeDtypeStruct((EP, H), jnp.float32)),
    mesh=_mesh,
    scratch_types=[
        pltpu.VMEM((8192,), jnp.int32),
        pltpu.VMEM((NCHUNK, CH), jnp.int32),
        pltpu.VMEM((NCHUNK, CH), jnp.int32),
        pltpu.VMEM((NCHUNK, CH), jnp.int32),
        pltpu.VMEM((NCHUNK, CH), jnp.int32),
        pltpu.VMEM((2, CH, H), jnp.float32),
        pltpu.VMEM_SHARED((KTAB,), jnp.int32),
        pltpu.SemaphoreType.DMA,
        pltpu.SemaphoreType.DMA,
        pltpu.SemaphoreType.DMA,
    ],
)
def _sc_prep(xw_hbm, eb_hbm, srcp_hbm, dstp_hbm, sidx_hbm, gidx_hbm, h0_hbm,
             fill_v, s_v, d_v, k_v, r_v, hbuf, table, sem_a, sem_b, sem_z):
    cid = lax.axis_index("c")
    sid = lax.axis_index("s")
    base = sid * EPT

    @pl.when(cid == 0)
    def _():
        ins = [pltpu.async_copy(srcp_hbm.at[pl.ds(base + c * CH, CH)],
                                s_v.at[c], sem_z) for c in range(NCHUNK)]
        pltpu.sync_copy(dstp_hbm.at[pl.ds(base, EPT)], fill_v.at[pl.ds(0, EPT)])
        pltpu.sync_copy(fill_v.at[pl.ds(0, EPT)], sidx_hbm.at[0, pl.ds(base, EPT)])
        for dsc in ins:
            dsc.wait()
        sems = (sem_a, sem_b)
        out = [None, None]
        for c in range(NCHUNK):
            sl = c % 2
            if out[sl] is not None:
                out[sl].wait()
            pltpu.async_copy(eb_hbm.at[pl.ds(base + c * CH, CH)], hbuf.at[sl],
                             sems[sl]).wait()
            pltpu.async_copy(xw_hbm.at[s_v.at[c]], hbuf.at[sl], sems[sl],
                             add=True).wait()
            for r in range(CH):
                for j in range(H // 16):
                    cs = pl.ds(j * 16, 16)
                    hbuf[sl, r, cs] = jnp.maximum(hbuf[sl, r, cs], 0.0)
            out[sl] = pltpu.async_copy(hbuf.at[sl],
                                       h0_hbm.at[pl.ds(base + c * CH, CH)],
                                       sems[sl])
        pltpu.sync_copy(srcp_hbm.at[pl.ds(base, EPT)], fill_v.at[pl.ds(0, EPT)])
        pltpu.sync_copy(fill_v.at[pl.ds(0, EPT)], gidx_hbm.at[0, pl.ds(base, EPT)])
        for d2 in out:
            if d2 is not None:
                d2.wait()

    @pl.when(cid == 1)
    def _():
        ins = []
        for c in range(NCHUNK):
            ins.append(pltpu.async_copy(srcp_hbm.at[pl.ds(base + c * CH, CH)],
                                        s_v.at[c], sem_a))
            ins.append(pltpu.async_copy(dstp_hbm.at[pl.ds(base + c * CH, CH)],
                                        d_v.at[c], sem_a))
        neg16 = jnp.full((16,), -1, jnp.int32)
        for t in range(8192 // 16):
            fill_v[pl.ds(t * 16, 16)] = neg16
        zs = [pltpu.async_copy(
                  fill_v, table.at[pl.ds(sid * (KTAB // NSUB) + z * 8192, 8192)],
                  sem_b) for z in range(KTAB // NSUB // 8192)]
        for dsc in ins:
            dsc.wait()
        for c in range(NCHUNK):
            for j in range(CH // 16):
                sl = pl.ds(j * 16, 16)
                k_v[c, sl] = s_v[c, sl] * N + d_v[c, sl]
                ids = lax.iota(jnp.int32, 16) + (base + c * CH + j * 16)
                r_v[c, sl] = jnp.minimum(ids, GTAB - 2)
        for dsc in zs:
            dsc.wait()
        plsc.subcore_barrier()
        sc = [pltpu.async_copy(r_v.at[c], table.at[k_v.at[c]], sem_a)
              for c in range(NCHUNK)]
        for dsc in sc:
            dsc.wait()
        plsc.subcore_barrier()
        ga = [pltpu.async_copy(table.at[k_v.at[c]], r_v.at[c], sem_a)
              for c in range(NCHUNK)]
        for dsc in ga:
            dsc.wait()
        outs = [pltpu.async_copy(r_v.at[c], sidx_hbm.at[1, pl.ds(base + c * CH, CH)],
                                 sem_b) for c in range(NCHUNK)]
        for c in range(NCHUNK):
            for j in range(CH // 16):
                sl = pl.ds(j * 16, 16)
                k_v[c, sl] = d_v[c, sl] * N + s_v[c, sl]
        for dsc in outs:
            dsc.wait()
        gb = [pltpu.async_copy(table.at[k_v.at[c]], r_v.at[c], sem_a)
              for c in range(NCHUNK)]
        for dsc in gb:
            dsc.wait()
        for c in range(NCHUNK):
            for j in range(CH // 16):
                sl = pl.ds(j * 16, 16)
                rv = r_v[c, sl]
                r_v[c, sl] = jnp.where(rv < 0, GTAB - 1, rv)
        outs = [pltpu.async_copy(r_v.at[c], gidx_hbm.at[1, pl.ds(base + c * CH, CH)],
                                 sem_b) for c in range(NCHUNK)]
        for dsc in outs:
            dsc.wait()


@functools.partial(
    pl.kernel,
    out_type=jax.ShapeDtypeStruct((2, EP, H), jnp.float32),
    mesh=_mesh,
    scratch_types=[
        pltpu.VMEM((NCHUNK, CH), jnp.int32),
        pltpu.VMEM((NCHUNK, CH), jnp.int32),
        pltpu.VMEM((2, CH, H), jnp.float32),
        pltpu.VMEM((CH, H), jnp.float32),
        pltpu.VMEM_SHARED((GTAB, H), jnp.float32),
        pltpu.SemaphoreType.DMA,
        pltpu.SemaphoreType.DMA,
        pltpu.SemaphoreType.DMA,
    ],
)
def _sc_round(h_hbm, sidx_hbm, gidx_hbm, m_hbm, siv, giv, hbuf, zbuf, table,
              sem_a, sem_b, sem_z):
    cid = lax.axis_index("c")
    sid = lax.axis_index("s")
    base = sid * EPT
    sems = (sem_a, sem_b)
    ins = []
    for c in range(NCHUNK):
        ins.append(pltpu.async_copy(sidx_hbm.at[cid, pl.ds(base + c * CH, CH)],
                                    siv.at[c], sem_z))
        ins.append(pltpu.async_copy(gidx_hbm.at[cid, pl.ds(base + c * CH, CH)],
                                    giv.at[c], sem_z))
    ld = [pltpu.async_copy(h_hbm.at[pl.ds(base + c * CH, CH)], hbuf.at[c],
                           sems[c]) for c in range(2)]
    _zero_rows(zbuf)
    zs = [pltpu.async_copy(zbuf, table.at[pl.ds(sid * (GTAB // NSUB) + z * CH, CH)],
                           sem_z) for z in range(GTAB // NSUB // CH)]
    for dsc in ins:
        dsc.wait()
    for dsc in zs:
        dsc.wait()
    ld2 = pltpu.async_copy(h_hbm.at[pl.ds(base + 2 * CH, CH)], zbuf, sem_z)
    for dsc in ld:
        dsc.wait()
    plsc.subcore_barrier()
    sc0 = pltpu.async_copy(hbuf.at[0], table.at[siv.at[0]], sem_a, add=True)
    sc1 = pltpu.async_copy(hbuf.at[1], table.at[siv.at[1]], sem_b, add=True)
    ld2.wait()
    sc2 = pltpu.async_copy(zbuf, table.at[siv.at[2]], sem_z, add=True)
    sc0.wait()
    pltpu.async_copy(h_hbm.at[pl.ds(base + 3 * CH, CH)], hbuf.at[0], sem_a).wait()
    sc3 = pltpu.async_copy(hbuf.at[0], table.at[siv.at[3]], sem_a, add=True)
    sc1.wait()
    sc2.wait()
    sc3.wait()
    plsc.subcore_barrier()
    g0 = pltpu.async_copy(table.at[giv.at[0]], hbuf.at[0], sem_a)
    g1 = pltpu.async_copy(table.at[giv.at[1]], hbuf.at[1], sem_b)
    g2 = pltpu.async_copy(table.at[giv.at[2]], zbuf, sem_z)
    g0.wait()
    o0 = pltpu.async_copy(hbuf.at[0], m_hbm.at[cid, pl.ds(base, CH)], sem_a)
    g1.wait()
    o1 = pltpu.async_copy(hbuf.at[1], m_hbm.at[cid, pl.ds(base + CH, CH)], sem_b)
    g2.wait()
    o2 = pltpu.async_copy(zbuf, m_hbm.at[cid, pl.ds(base + 2 * CH, CH)], sem_z)
    o0.wait()
    pltpu.async_copy(table.at[giv.at[3]], hbuf.at[0], sem_a).wait()
    o3 = pltpu.async_copy(hbuf.at[0], m_hbm.at[cid, pl.ds(base + 3 * CH, CH)],
                          sem_a)
    o1.wait()
    o2.wait()
    o3.wait()


@functools.partial(
    pl.kernel,
    out_type=jax.ShapeDtypeStruct((NTAB, H), jnp.float32),
    mesh=_mesh,
    scratch_types=[
        pltpu.VMEM((NCHUNK, CH), jnp.int32),
        pltpu.VMEM((2, CH, H), jnp.float32),
        pltpu.VMEM((CH, H), jnp.float32),
        pltpu.VMEM_SHARED((NTAB, H), jnp.float32),
        pltpu.SemaphoreType.DMA,
        pltpu.SemaphoreType.DMA,
        pltpu.SemaphoreType.DMA,
    ],
)
def _sc_final(hw_hbm, dstp_hbm, xa_hbm, out_hbm, iv, hbuf, zbuf, table,
              sem_a, sem_b, sem_z):
    cid = lax.axis_index("c")
    sid = lax.axis_index("s")
    base = sid * EPT
    sems = (sem_a, sem_b)
    ins = [pltpu.async_copy(dstp_hbm.at[pl.ds(base + c * CH, CH)], iv.at[c],
                            sem_z) for c in range(NCHUNK)]
    ld = [pltpu.async_copy(hw_hbm.at[pl.ds(base + c * CH, CH)], hbuf.at[c],
                           sems[c]) for c in range(2)]
    _zero_rows(zbuf)
    rpt = NTAB // NSUB
    z = pltpu.async_copy(zbuf.at[pl.ds(0, rpt)], table.at[pl.ds(sid * rpt, rpt)],
                         sem_z)
    for dsc in ins:
        dsc.wait()
    z.wait()
    ld2 = pltpu.async_copy(hw_hbm.at[pl.ds(base + 2 * CH, CH)], zbuf, sem_z)
    for dsc in ld:
        dsc.wait()
    plsc.subcore_barrier()
    sc0 = pltpu.async_copy(hbuf.at[0], table.at[iv.at[0]], sem_a, add=True)
    sc1 = pltpu.async_copy(hbuf.at[1], table.at[iv.at[1]], sem_b, add=True)
    ld2.wait()
    sc2 = pltpu.async_copy(zbuf, table.at[iv.at[2]], sem_z, add=True)
    sc0.wait()
    pltpu.async_copy(hw_hbm.at[pl.ds(base + 3 * CH, CH)], hbuf.at[0], sem_a).wait()
    sc3 = pltpu.async_copy(hbuf.at[0], table.at[iv.at[3]], sem_a, add=True)
    sc1.wait()
    sc2.wait()
    sc3.wait()
    plsc.subcore_barrier()
    rpo = 512 // NSUB
    row0 = cid * 512 + sid * rpo
    pltpu.sync_copy(xa_hbm.at[pl.ds(row0, rpo)], hbuf.at[0, pl.ds(0, rpo)])
    pltpu.sync_copy(table.at[pl.ds(row0, rpo)], hbuf.at[1, pl.ds(0, rpo)])

    def orow(r, _):
        for c in range(H // 16):
            sl = pl.ds(c * 16, 16)
            hbuf[0, r, sl] = jnp.maximum(hbuf[0, r, sl] + hbuf[1, r, sl], 0.0)
        return 0

    lax.fori_loop(0, rpo, orow, 0)
    pltpu.sync_copy(hbuf.at[0, pl.ds(0, rpo)], out_hbm.at[pl.ds(row0, rpo)])


def _eb_body(ea_ref, w2_ref, b_ref, o_ref):
    o_ref[...] = (jnp.dot(ea_ref[...], w2_ref[...],
                          preferred_element_type=jnp.float32) + b_ref[...])


def _tc_eb(ea, w2t, b):
    return pl.pallas_call(
        _eb_body,
        grid=(EP // 1024,),
        in_specs=[
            pl.BlockSpec((1024, DE), lambda i: (i, 0)),
            pl.BlockSpec((DE, H), lambda i: (0, 0)),
            pl.BlockSpec((1, H), lambda i: (0, 0)),
        ],
        out_specs=pl.BlockSpec((1024, H), lambda i: (i, 0)),
        out_shape=jax.ShapeDtypeStruct((EP, H), jnp.float32),
    )(ea, w2t, b)


def _round_body(m_ref, h0_ref, w_ref, b_ref, o_ref):
    mm = m_ref[0] - m_ref[1]
    acc = (h0_ref[...]
           + jnp.dot(mm, w_ref[...], preferred_element_type=jnp.float32)
           + b_ref[...])
    o_ref[...] = jnp.maximum(acc, 0.0)


def _tc_round(m, h0, wt, b):
    return pl.pallas_call(
        _round_body,
        grid=(EP // 1024,),
        in_specs=[
            pl.BlockSpec((2, 1024, H), lambda i: (0, i, 0)),
            pl.BlockSpec((1024, H), lambda i: (i, 0)),
            pl.BlockSpec((H, H), lambda i: (0, 0)),
            pl.BlockSpec((1, H), lambda i: (0, 0)),
        ],
        out_specs=pl.BlockSpec((1024, H), lambda i: (i, 0)),
        out_shape=jax.ShapeDtypeStruct((EP, H), jnp.float32),
    )(m, h0, wt, b)


def _round_final_body(m_ref, h0_ref, w_ref, b_ref, w2_ref, o_ref):
    mm = m_ref[0] - m_ref[1]
    h3 = jnp.maximum(
        h0_ref[...]
        + jnp.dot(mm, w_ref[...], preferred_element_type=jnp.float32)
        + b_ref[...], 0.0)
    o_ref[...] = jnp.dot(h3, w2_ref[...], preferred_element_type=jnp.float32)


def _tc_round_final(m, h0, wt, b, wn2t):
    return pl.pallas_call(
        _round_final_body,
        grid=(EP // 1024,),
        in_specs=[
            pl.BlockSpec((2, 1024, H), lambda i: (0, i, 0)),
            pl.BlockSpec((1024, H), lambda i: (i, 0)),
            pl.BlockSpec((H, H), lambda i: (0, 0)),
            pl.BlockSpec((1, H), lambda i: (0, 0)),
            pl.BlockSpec((H, H), lambda i: (0, 0)),
        ],
        out_specs=pl.BlockSpec((1024, H), lambda i: (i, 0)),
        out_shape=jax.ShapeDtypeStruct((EP, H), jnp.float32),
    )(m, h0, wt, b, wn2t)


def _pre_body(x_ref, w1_ref, wn_ref, b_ref, xw_ref, xa_ref):
    xw_ref[...] = jnp.dot(x_ref[...], w1_ref[...],
                          preferred_element_type=jnp.float32)
    xa_ref[...] = (jnp.dot(x_ref[...], wn_ref[...],
                           preferred_element_type=jnp.float32) + b_ref[...])


def _tc_pre(xp, w1t, wn1t, b):
    return pl.pallas_call(
        _pre_body,
        grid=(NTAB // 512,),
        in_specs=[
            pl.BlockSpec((512, D), lambda i: (i, 0)),
            pl.BlockSpec((D, H), lambda i: (0, 0)),
            pl.BlockSpec((D, H), lambda i: (0, 0)),
            pl.BlockSpec((1, H), lambda i: (0, 0)),
        ],
        out_specs=[
            pl.BlockSpec((512, H), lambda i: (i, 0)),
            pl.BlockSpec((512, H), lambda i: (i, 0)),
        ],
        out_shape=[
            jax.ShapeDtypeStruct((NTAB, H), jnp.float32),
            jax.ShapeDtypeStruct((NTAB, H), jnp.float32),
        ],
    )(xp, w1t, wn1t, b)


def kernel(x, edge_index, edge_attr, W_edge_init, b_edge_init, W_msg, b_msg,
           W_node, b_node):
    src = edge_index[0]
    dst = edge_index[1]

    pad = EP - E
    srcp = jnp.concatenate([src, jnp.full((pad,), N, jnp.int32)])
    dstp = jnp.concatenate([dst, jnp.full((pad,), N, jnp.int32)])
    eap = jnp.pad(edge_attr, ((0, pad), (0, 0)))

    w1t = W_edge_init[:, :D].T
    w2t = W_edge_init[:, D:].T
    wmt = W_msg.T
    wn1t = W_node[:, :D].T
    wn2t = W_node[:, D:].T
    be = b_edge_init.reshape(1, H)
    bm = b_msg.reshape(1, H)
    bn = b_node.reshape(1, H)

    xw, xa = _tc_pre(x, w1t, wn1t, bn)
    eb = _tc_eb(eap, w2t, be)
    sidx, gidx, h0 = _sc_prep(xw, eb, srcp, dstp)
    h = h0
    for t in range(T - 1):
        m = _sc_round(h, sidx, gidx)
        h = _tc_round(m, h0, wmt, bm)
    m = _sc_round(h, sidx, gidx)
    hw = _tc_round_final(m, h0, wmt, bm, wn2t)
    out = _sc_final(hw, dstp, xa)
    return out[:N]


# speedup vs baseline: 1.0746x; 1.0675x over previous
import functools

import jax
import jax.numpy as jnp
from jax import lax
from jax.experimental import pallas as pl
from jax.experimental.pallas import tpu as pltpu
from jax.experimental.pallas import tpu_sc as plsc

N = 1000
E = 8000
D = 128
DE = 16
H = 128
T = 3

EP = 8192
CH = 128
NTAB = 1024
GTAB = 8192
KTAB = 1 << 20
NSUB = 16
EPT = EP // NSUB
NCHUNK = EPT // CH
NC2 = EP // (32 * CH)

_mesh = plsc.VectorSubcoreMesh(core_axis_name="c", subcore_axis_name="s")


def _zero_rows(buf):
    z = jnp.zeros((16,), jnp.float32)
    for r in range(CH):
        for c in range(H // 16):
            buf[r, pl.ds(c * 16, 16)] = z


@functools.partial(
    pl.kernel,
    out_type=(jax.ShapeDtypeStruct((2, EP), jnp.int32),
              jax.ShapeDtypeStruct((2, EP), jnp.int32),
              jax.ShapeDtypeStruct((EP, H), jnp.float32)),
    mesh=_mesh,
    scratch_types=[
        pltpu.VMEM((8192,), jnp.int32),
        pltpu.VMEM((NCHUNK, CH), jnp.int32),
        pltpu.VMEM((NCHUNK, CH), jnp.int32),
        pltpu.VMEM((NCHUNK, CH), jnp.int32),
        pltpu.VMEM((NCHUNK, CH), jnp.int32),
        pltpu.VMEM((2, CH, H), jnp.float32),
        pltpu.VMEM_SHARED((KTAB,), jnp.int32),
        pltpu.SemaphoreType.DMA,
        pltpu.SemaphoreType.DMA,
        pltpu.SemaphoreType.DMA,
    ],
)
def _sc_prep(xw_hbm, eb_hbm, srcp_hbm, dstp_hbm, sidx_hbm, gidx_hbm, h0_hbm,
             fill_v, s_v, d_v, k_v, r_v, hbuf, table, sem_a, sem_b, sem_z):
    cid = lax.axis_index("c")
    sid = lax.axis_index("s")
    base = sid * EPT

    @pl.when(cid == 0)
    def _():
        ins = [pltpu.async_copy(srcp_hbm.at[pl.ds(base + c * CH, CH)],
                                s_v.at[c], sem_z) for c in range(NCHUNK)]
        pltpu.sync_copy(dstp_hbm.at[pl.ds(base, EPT)], fill_v.at[pl.ds(0, EPT)])
        pltpu.sync_copy(fill_v.at[pl.ds(0, EPT)], sidx_hbm.at[0, pl.ds(base, EPT)])
        for dsc in ins:
            dsc.wait()
        sems = (sem_a, sem_b)

        def _relu(sl):
            for r in range(CH):
                for j in range(H // 16):
                    cs = pl.ds(j * 16, 16)
                    hbuf[sl, r, cs] = jnp.maximum(hbuf[sl, r, cs], 0.0)

        def _ebld(c, sl):
            return pltpu.async_copy(eb_hbm.at[pl.ds(base + c * CH, CH)],
                                    hbuf.at[sl], sems[sl])

        def _gadd(c, sl):
            return pltpu.async_copy(xw_hbm.at[s_v.at[c]], hbuf.at[sl],
                                    sems[sl], add=True)

        def _h0out(c, sl):
            return pltpu.async_copy(hbuf.at[sl],
                                    h0_hbm.at[pl.ds(base + c * CH, CH)],
                                    sems[sl])

        e0 = _ebld(0, 0)
        e1 = _ebld(1, 1)
        e0.wait()
        g0 = _gadd(0, 0)
        e1.wait()
        g1 = _gadd(1, 1)
        g0.wait()
        _relu(0)
        o0 = _h0out(0, 0)
        g1.wait()
        _relu(1)
        o1 = _h0out(1, 1)
        o0.wait()
        e2 = _ebld(2, 0)
        e2.wait()
        g2 = _gadd(2, 0)
        o1.wait()
        e3 = _ebld(3, 1)
        e3.wait()
        g3 = _gadd(3, 1)
        g2.wait()
        _relu(0)
        o2 = _h0out(2, 0)
        g3.wait()
        _relu(1)
        o3 = _h0out(3, 1)
        pltpu.sync_copy(srcp_hbm.at[pl.ds(base, EPT)], fill_v.at[pl.ds(0, EPT)])
        pltpu.sync_copy(fill_v.at[pl.ds(0, EPT)], gidx_hbm.at[0, pl.ds(base, EPT)])
        o2.wait()
        o3.wait()

    @pl.when(cid == 1)
    def _():
        ins = []
        for c in range(NCHUNK):
            ins.append(pltpu.async_copy(srcp_hbm.at[pl.ds(base + c * CH, CH)],
                                        s_v.at[c], sem_a))
            ins.append(pltpu.async_copy(dstp_hbm.at[pl.ds(base + c * CH, CH)],
                                        d_v.at[c], sem_a))
        neg16 = jnp.full((16,), -1, jnp.int32)
        for t in range(8192 // 16):
            fill_v[pl.ds(t * 16, 16)] = neg16
        zs = [pltpu.async_copy(
                  fill_v, table.at[pl.ds(sid * (KTAB // NSUB) + z * 8192, 8192)],
                  sem_b) for z in range(KTAB // NSUB // 8192)]
        for dsc in ins:
            dsc.wait()
        for c in range(NCHUNK):
            for j in range(CH // 16):
                sl = pl.ds(j * 16, 16)
                k_v[c, sl] = s_v[c, sl] * N + d_v[c, sl]
                ids = lax.iota(jnp.int32, 16) + (base + c * CH + j * 16)
                r_v[c, sl] = jnp.minimum(ids, GTAB - 2)
        for dsc in zs:
            dsc.wait()
        plsc.subcore_barrier()
        sc = [pltpu.async_copy(r_v.at[c], table.at[k_v.at[c]], sem_a)
              for c in range(NCHUNK)]
        for dsc in sc:
            dsc.wait()
        plsc.subcore_barrier()
        ga = [pltpu.async_copy(table.at[k_v.at[c]], r_v.at[c], sem_a)
              for c in range(NCHUNK)]
        for dsc in ga:
            dsc.wait()
        outs = [pltpu.async_copy(r_v.at[c], sidx_hbm.at[1, pl.ds(base + c * CH, CH)],
                                 sem_b) for c in range(NCHUNK)]
        for c in range(NCHUNK):
            for j in range(CH // 16):
                sl = pl.ds(j * 16, 16)
                k_v[c, sl] = d_v[c, sl] * N + s_v[c, sl]
        for dsc in outs:
            dsc.wait()
        gb = [pltpu.async_copy(table.at[k_v.at[c]], r_v.at[c], sem_a)
              for c in range(NCHUNK)]
        for dsc in gb:
            dsc.wait()
        for c in range(NCHUNK):
            for j in range(CH // 16):
                sl = pl.ds(j * 16, 16)
                rv = r_v[c, sl]
                r_v[c, sl] = jnp.where(rv < 0, GTAB - 1, rv)
        outs = [pltpu.async_copy(r_v.at[c], gidx_hbm.at[1, pl.ds(base + c * CH, CH)],
                                 sem_b) for c in range(NCHUNK)]
        for dsc in outs:
            dsc.wait()


@functools.partial(
    pl.kernel,
    out_type=jax.ShapeDtypeStruct((2, EP, H), jnp.float32),
    mesh=_mesh,
    scratch_types=[
        pltpu.VMEM((NCHUNK, CH), jnp.int32),
        pltpu.VMEM((NCHUNK, CH), jnp.int32),
        pltpu.VMEM((2, CH, H), jnp.float32),
        pltpu.VMEM((CH, H), jnp.float32),
        pltpu.VMEM_SHARED((GTAB, H), jnp.float32),
        pltpu.SemaphoreType.DMA,
        pltpu.SemaphoreType.DMA,
        pltpu.SemaphoreType.DMA,
    ],
)
def _sc_round(h_hbm, sidx_hbm, gidx_hbm, m_hbm, siv, giv, hbuf, zbuf, table,
              sem_a, sem_b, sem_z):
    cid = lax.axis_index("c")
    sid = lax.axis_index("s")
    base = sid * EPT
    sems = (sem_a, sem_b)
    ins = []
    for c in range(NCHUNK):
        ins.append(pltpu.async_copy(sidx_hbm.at[cid, pl.ds(base + c * CH, CH)],
                                    siv.at[c], sem_z))
        ins.append(pltpu.async_copy(gidx_hbm.at[cid, pl.ds(base + c * CH, CH)],
                                    giv.at[c], sem_z))
    ld = [pltpu.async_copy(h_hbm.at[pl.ds(base + c * CH, CH)], hbuf.at[c],
                           sems[c]) for c in range(2)]
    _zero_rows(zbuf)
    zs = [pltpu.async_copy(zbuf, table.at[pl.ds(sid * (GTAB // NSUB) + z * CH, CH)],
                           sem_z) for z in range(GTAB // NSUB // CH)]
    for dsc in ins:
        dsc.wait()
    for dsc in zs:
        dsc.wait()
    ld2 = pltpu.async_copy(h_hbm.at[pl.ds(base + 2 * CH, CH)], zbuf, sem_z)
    for dsc in ld:
        dsc.wait()
    plsc.subcore_barrier()
    sc0 = pltpu.async_copy(hbuf.at[0], table.at[siv.at[0]], sem_a, add=True)
    sc1 = pltpu.async_copy(hbuf.at[1], table.at[siv.at[1]], sem_b, add=True)
    ld2.wait()
    sc2 = pltpu.async_copy(zbuf, table.at[siv.at[2]], sem_z, add=True)
    sc0.wait()
    pltpu.async_copy(h_hbm.at[pl.ds(base + 3 * CH, CH)], hbuf.at[0], sem_a).wait()
    sc3 = pltpu.async_copy(hbuf.at[0], table.at[siv.at[3]], sem_a, add=True)
    sc1.wait()
    sc2.wait()
    sc3.wait()
    plsc.subcore_barrier()
    g0 = pltpu.async_copy(table.at[giv.at[0]], hbuf.at[0], sem_a)
    g1 = pltpu.async_copy(table.at[giv.at[1]], hbuf.at[1], sem_b)
    g2 = pltpu.async_copy(table.at[giv.at[2]], zbuf, sem_z)
    g0.wait()
    o0 = pltpu.async_copy(hbuf.at[0], m_hbm.at[cid, pl.ds(base, CH)], sem_a)
    g1.wait()
    o1 = pltpu.async_copy(hbuf.at[1], m_hbm.at[cid, pl.ds(base + CH, CH)], sem_b)
    g2.wait()
    o2 = pltpu.async_copy(zbuf, m_hbm.at[cid, pl.ds(base + 2 * CH, CH)], sem_z)
    o0.wait()
    pltpu.async_copy(table.at[giv.at[3]], hbuf.at[0], sem_a).wait()
    o3 = pltpu.async_copy(hbuf.at[0], m_hbm.at[cid, pl.ds(base + 3 * CH, CH)],
                          sem_a)
    o1.wait()
    o2.wait()
    o3.wait()


@functools.partial(
    pl.kernel,
    out_type=jax.ShapeDtypeStruct((NTAB, H), jnp.float32),
    mesh=_mesh,
    scratch_types=[
        pltpu.VMEM((NCHUNK, CH), jnp.int32),
        pltpu.VMEM((2, CH, H), jnp.float32),
        pltpu.VMEM((CH, H), jnp.float32),
        pltpu.VMEM_SHARED((NTAB, H), jnp.float32),
        pltpu.SemaphoreType.DMA,
        pltpu.SemaphoreType.DMA,
        pltpu.SemaphoreType.DMA,
    ],
)
def _sc_final(hw_hbm, dstp_hbm, xa_hbm, out_hbm, iv, hbuf, zbuf, table,
              sem_a, sem_b, sem_z):
    cid = lax.axis_index("c")
    sid = lax.axis_index("s")
    base = sid * EPT
    sems = (sem_a, sem_b)
    ins = [pltpu.async_copy(dstp_hbm.at[pl.ds(base + c * CH, CH)], iv.at[c],
                            sem_z) for c in range(NCHUNK)]
    ld = [pltpu.async_copy(hw_hbm.at[pl.ds(base + c * CH, CH)], hbuf.at[c],
                           sems[c]) for c in range(2)]
    _zero_rows(zbuf)
    rpt = NTAB // NSUB
    z = pltpu.async_copy(zbuf.at[pl.ds(0, rpt)], table.at[pl.ds(sid * rpt, rpt)],
                         sem_z)
    for dsc in ins:
        dsc.wait()
    z.wait()
    ld2 = pltpu.async_copy(hw_hbm.at[pl.ds(base + 2 * CH, CH)], zbuf, sem_z)
    for dsc in ld:
        dsc.wait()
    plsc.subcore_barrier()
    sc0 = pltpu.async_copy(hbuf.at[0], table.at[iv.at[0]], sem_a, add=True)
    sc1 = pltpu.async_copy(hbuf.at[1], table.at[iv.at[1]], sem_b, add=True)
    ld2.wait()
    sc2 = pltpu.async_copy(zbuf, table.at[iv.at[2]], sem_z, add=True)
    sc0.wait()
    pltpu.async_copy(hw_hbm.at[pl.ds(base + 3 * CH, CH)], hbuf.at[0], sem_a).wait()
    sc3 = pltpu.async_copy(hbuf.at[0], table.at[iv.at[3]], sem_a, add=True)
    sc1.wait()
    sc2.wait()
    sc3.wait()
    plsc.subcore_barrier()
    rpo = 512 // NSUB
    row0 = cid * 512 + sid * rpo
    pltpu.sync_copy(xa_hbm.at[pl.ds(row0, rpo)], hbuf.at[0, pl.ds(0, rpo)])
    pltpu.sync_copy(table.at[pl.ds(row0, rpo)], hbuf.at[1, pl.ds(0, rpo)])

    def orow(r, _):
        for c in range(H // 16):
            sl = pl.ds(c * 16, 16)
            hbuf[0, r, sl] = jnp.maximum(hbuf[0, r, sl] + hbuf[1, r, sl], 0.0)
        return 0

    lax.fori_loop(0, rpo, orow, 0)
    pltpu.sync_copy(hbuf.at[0, pl.ds(0, rpo)], out_hbm.at[pl.ds(row0, rpo)])


def _eb_body(ea_ref, w2_ref, b_ref, o_ref):
    o_ref[...] = (jnp.dot(ea_ref[...], w2_ref[...],
                          preferred_element_type=jnp.float32) + b_ref[...])


def _tc_eb(ea, w2t, b):
    return pl.pallas_call(
        _eb_body,
        grid=(EP // 1024,),
        in_specs=[
            pl.BlockSpec((1024, DE), lambda i: (i, 0)),
            pl.BlockSpec((DE, H), lambda i: (0, 0)),
            pl.BlockSpec((1, H), lambda i: (0, 0)),
        ],
        out_specs=pl.BlockSpec((1024, H), lambda i: (i, 0)),
        out_shape=jax.ShapeDtypeStruct((EP, H), jnp.float32),
    )(ea, w2t, b)


def _round_body(m_ref, h0_ref, w_ref, b_ref, o_ref):
    mm = m_ref[0] - m_ref[1]
    acc = (h0_ref[...]
           + jnp.dot(mm, w_ref[...], preferred_element_type=jnp.float32)
           + b_ref[...])
    o_ref[...] = jnp.maximum(acc, 0.0)


def _tc_round(m, h0, wt, b):
    return pl.pallas_call(
        _round_body,
        grid=(EP // 2048,),
        in_specs=[
            pl.BlockSpec((2, 2048, H), lambda i: (0, i, 0)),
            pl.BlockSpec((2048, H), lambda i: (i, 0)),
            pl.BlockSpec((H, H), lambda i: (0, 0)),
            pl.BlockSpec((1, H), lambda i: (0, 0)),
        ],
        out_specs=pl.BlockSpec((2048, H), lambda i: (i, 0)),
        out_shape=jax.ShapeDtypeStruct((EP, H), jnp.float32),
    )(m, h0, wt, b)


def _round_final_body(m_ref, h0_ref, w_ref, b_ref, w2_ref, o_ref):
    mm = m_ref[0] - m_ref[1]
    h3 = jnp.maximum(
        h0_ref[...]
        + jnp.dot(mm, w_ref[...], preferred_element_type=jnp.float32)
        + b_ref[...], 0.0)
    o_ref[...] = jnp.dot(h3, w2_ref[...], preferred_element_type=jnp.float32)


def _tc_round_final(m, h0, wt, b, wn2t):
    return pl.pallas_call(
        _round_final_body,
        grid=(EP // 2048,),
        in_specs=[
            pl.BlockSpec((2, 2048, H), lambda i: (0, i, 0)),
            pl.BlockSpec((2048, H), lambda i: (i, 0)),
            pl.BlockSpec((H, H), lambda i: (0, 0)),
            pl.BlockSpec((1, H), lambda i: (0, 0)),
            pl.BlockSpec((H, H), lambda i: (0, 0)),
        ],
        out_specs=pl.BlockSpec((2048, H), lambda i: (i, 0)),
        out_shape=jax.ShapeDtypeStruct((EP, H), jnp.float32),
    )(m, h0, wt, b, wn2t)


def _pre_body(x_ref, w1_ref, wn_ref, b_ref, xw_ref, xa_ref):
    xw_ref[...] = jnp.dot(x_ref[...], w1_ref[...],
                          preferred_element_type=jnp.float32)
    xa_ref[...] = (jnp.dot(x_ref[...], wn_ref[...],
                           preferred_element_type=jnp.float32) + b_ref[...])


def _tc_pre(xp, w1t, wn1t, b):
    return pl.pallas_call(
        _pre_body,
        grid=(NTAB // 512,),
        in_specs=[
            pl.BlockSpec((512, D), lambda i: (i, 0)),
            pl.BlockSpec((D, H), lambda i: (0, 0)),
            pl.BlockSpec((D, H), lambda i: (0, 0)),
            pl.BlockSpec((1, H), lambda i: (0, 0)),
        ],
        out_specs=[
            pl.BlockSpec((512, H), lambda i: (i, 0)),
            pl.BlockSpec((512, H), lambda i: (i, 0)),
        ],
        out_shape=[
            jax.ShapeDtypeStruct((NTAB, H), jnp.float32),
            jax.ShapeDtypeStruct((NTAB, H), jnp.float32),
        ],
    )(xp, w1t, wn1t, b)


def kernel(x, edge_index, edge_attr, W_edge_init, b_edge_init, W_msg, b_msg,
           W_node, b_node):
    src = edge_index[0]
    dst = edge_index[1]

    pad = EP - E
    srcp = jnp.concatenate([src, jnp.full((pad,), N, jnp.int32)])
    dstp = jnp.concatenate([dst, jnp.full((pad,), N, jnp.int32)])
    eap = jnp.pad(edge_attr, ((0, pad), (0, 0)))

    w1t = W_edge_init[:, :D].T
    w2t = W_edge_init[:, D:].T
    wmt = W_msg.T
    wn1t = W_node[:, :D].T
    wn2t = W_node[:, D:].T
    be = b_edge_init.reshape(1, H)
    bm = b_msg.reshape(1, H)
    bn = b_node.reshape(1, H)

    xw, xa = _tc_pre(x, w1t, wn1t, bn)
    eb = _tc_eb(eap, w2t, be)
    sidx, gidx, h0 = _sc_prep(xw, eb, srcp, dstp)
    h = h0
    for t in range(T - 1):
        m = _sc_round(h, sidx, gidx)
        h = _tc_round(m, h0, wmt, bm)
    m = _sc_round(h, sidx, gidx)
    hw = _tc_round_final(m, h0, wmt, bm, wn2t)
    out = _sc_final(hw, dstp, xa)
    return out[:N]


# speedup vs baseline: 1.1009x; 1.0245x over previous
import functools

import jax
import jax.numpy as jnp
from jax import lax
from jax.experimental import pallas as pl
from jax.experimental.pallas import tpu as pltpu
from jax.experimental.pallas import tpu_sc as plsc

N = 1000
E = 8000
D = 128
DE = 16
H = 128
T = 3

EP = 8192
CH = 128
NTAB = 1024
GTAB = 8192
KTAB = 1 << 20
NSUB = 16
EPT = EP // NSUB
NCHUNK = EPT // CH
NC2 = EP // (32 * CH)

_mesh = plsc.VectorSubcoreMesh(core_axis_name="c", subcore_axis_name="s")


def _zero_rows(buf):
    z = jnp.zeros((16,), jnp.float32)
    for r in range(CH):
        for c in range(H // 16):
            buf[r, pl.ds(c * 16, 16)] = z


@functools.partial(
    pl.kernel,
    out_type=(jax.ShapeDtypeStruct((2, EP), jnp.int32),
              jax.ShapeDtypeStruct((2, EP), jnp.int32),
              jax.ShapeDtypeStruct((EP, H), jnp.float32)),
    mesh=_mesh,
    scratch_types=[
        pltpu.VMEM((8192,), jnp.int32),
        pltpu.VMEM((NCHUNK, CH), jnp.int32),
        pltpu.VMEM((NCHUNK, CH), jnp.int32),
        pltpu.VMEM((NCHUNK, CH), jnp.int32),
        pltpu.VMEM((NCHUNK, CH), jnp.int32),
        pltpu.VMEM((2, CH, H), jnp.float32),
        pltpu.VMEM_SHARED((KTAB,), jnp.int32),
        pltpu.SemaphoreType.DMA,
        pltpu.SemaphoreType.DMA,
        pltpu.SemaphoreType.DMA,
    ],
)
def _sc_prep(xw_hbm, eb_hbm, srcp_hbm, dstp_hbm, sidx_hbm, gidx_hbm, h0_hbm,
             fill_v, s_v, d_v, k_v, r_v, hbuf, table, sem_a, sem_b, sem_z):
    cid = lax.axis_index("c")
    sid = lax.axis_index("s")
    base = sid * EPT

    @pl.when(cid == 0)
    def _():
        ins = [pltpu.async_copy(srcp_hbm.at[pl.ds(base + c * CH, CH)],
                                s_v.at[c], sem_z) for c in range(NCHUNK)]
        pltpu.sync_copy(dstp_hbm.at[pl.ds(base, EPT)], fill_v.at[pl.ds(0, EPT)])
        pltpu.sync_copy(fill_v.at[pl.ds(0, EPT)], sidx_hbm.at[0, pl.ds(base, EPT)])
        for dsc in ins:
            dsc.wait()
        sems = (sem_a, sem_b)

        def _relu(sl):
            for r in range(CH):
                for j in range(H // 16):
                    cs = pl.ds(j * 16, 16)
                    hbuf[sl, r, cs] = jnp.maximum(hbuf[sl, r, cs], 0.0)

        def _ebld(c, sl):
            return pltpu.async_copy(eb_hbm.at[pl.ds(base + c * CH, CH)],
                                    hbuf.at[sl], sems[sl])

        def _gadd(c, sl):
            return pltpu.async_copy(xw_hbm.at[s_v.at[c]], hbuf.at[sl],
                                    sems[sl], add=True)

        def _h0out(c, sl):
            return pltpu.async_copy(hbuf.at[sl],
                                    h0_hbm.at[pl.ds(base + c * CH, CH)],
                                    sems[sl])

        e0 = _ebld(0, 0)
        e1 = _ebld(1, 1)
        e0.wait()
        g0 = _gadd(0, 0)
        e1.wait()
        g1 = _gadd(1, 1)
        g0.wait()
        _relu(0)
        o0 = _h0out(0, 0)
        g1.wait()
        _relu(1)
        o1 = _h0out(1, 1)
        o0.wait()
        e2 = _ebld(2, 0)
        e2.wait()
        g2 = _gadd(2, 0)
        o1.wait()
        e3 = _ebld(3, 1)
        e3.wait()
        g3 = _gadd(3, 1)
        g2.wait()
        _relu(0)
        o2 = _h0out(2, 0)
        g3.wait()
        _relu(1)
        o3 = _h0out(3, 1)
        pltpu.sync_copy(srcp_hbm.at[pl.ds(base, EPT)], fill_v.at[pl.ds(0, EPT)])
        pltpu.sync_copy(fill_v.at[pl.ds(0, EPT)], gidx_hbm.at[0, pl.ds(base, EPT)])
        o2.wait()
        o3.wait()

    @pl.when(cid == 1)
    def _():
        ins = []
        for c in range(NCHUNK):
            ins.append(pltpu.async_copy(srcp_hbm.at[pl.ds(base + c * CH, CH)],
                                        s_v.at[c], sem_a))
            ins.append(pltpu.async_copy(dstp_hbm.at[pl.ds(base + c * CH, CH)],
                                        d_v.at[c], sem_a))
        neg16 = jnp.full((16,), -1, jnp.int32)
        for t in range(8192 // 16):
            fill_v[pl.ds(t * 16, 16)] = neg16
        zs = [pltpu.async_copy(
                  fill_v, table.at[pl.ds(sid * (KTAB // NSUB) + z * 8192, 8192)],
                  sem_b) for z in range(KTAB // NSUB // 8192)]
        for dsc in ins:
            dsc.wait()
        for c in range(NCHUNK):
            for j in range(CH // 16):
                sl = pl.ds(j * 16, 16)
                k_v[c, sl] = s_v[c, sl] * N + d_v[c, sl]
                ids = lax.iota(jnp.int32, 16) + (base + c * CH + j * 16)
                r_v[c, sl] = jnp.minimum(ids, GTAB - 2)
        for dsc in zs:
            dsc.wait()
        plsc.subcore_barrier()
        sc = [pltpu.async_copy(r_v.at[c], table.at[k_v.at[c]], sem_a)
              for c in range(NCHUNK)]
        for dsc in sc:
            dsc.wait()
        plsc.subcore_barrier()
        ga = [pltpu.async_copy(table.at[k_v.at[c]], r_v.at[c], sem_a)
              for c in range(NCHUNK)]
        for dsc in ga:
            dsc.wait()
        outs = [pltpu.async_copy(r_v.at[c], sidx_hbm.at[1, pl.ds(base + c * CH, CH)],
                                 sem_b) for c in range(NCHUNK)]
        for c in range(NCHUNK):
            for j in range(CH // 16):
                sl = pl.ds(j * 16, 16)
                k_v[c, sl] = d_v[c, sl] * N + s_v[c, sl]
        for dsc in outs:
            dsc.wait()
        gb = [pltpu.async_copy(table.at[k_v.at[c]], r_v.at[c], sem_a)
              for c in range(NCHUNK)]
        for dsc in gb:
            dsc.wait()
        for c in range(NCHUNK):
            for j in range(CH // 16):
                sl = pl.ds(j * 16, 16)
                rv = r_v[c, sl]
                r_v[c, sl] = jnp.where(rv < 0, GTAB - 1, rv)
        outs = [pltpu.async_copy(r_v.at[c], gidx_hbm.at[1, pl.ds(base + c * CH, CH)],
                                 sem_b) for c in range(NCHUNK)]
        for dsc in outs:
            dsc.wait()


@functools.partial(
    pl.kernel,
    out_type=jax.ShapeDtypeStruct((2, EP, H), jnp.float32),
    mesh=_mesh,
    scratch_types=[
        pltpu.VMEM((NCHUNK, CH), jnp.int32),
        pltpu.VMEM((NCHUNK, CH), jnp.int32),
        pltpu.VMEM((2, CH, H), jnp.float32),
        pltpu.VMEM((CH, H), jnp.float32),
        pltpu.VMEM_SHARED((GTAB, H), jnp.float32),
        pltpu.SemaphoreType.DMA,
        pltpu.SemaphoreType.DMA,
        pltpu.SemaphoreType.DMA,
    ],
)
def _sc_round(h_hbm, sidx_hbm, gidx_hbm, m_hbm, siv, giv, hbuf, zbuf, table,
              sem_a, sem_b, sem_z):
    cid = lax.axis_index("c")
    sid = lax.axis_index("s")
    base = sid * EPT
    sems = (sem_a, sem_b)
    ins = []
    for c in range(NCHUNK):
        ins.append(pltpu.async_copy(sidx_hbm.at[cid, pl.ds(base + c * CH, CH)],
                                    siv.at[c], sem_z))
        ins.append(pltpu.async_copy(gidx_hbm.at[cid, pl.ds(base + c * CH, CH)],
                                    giv.at[c], sem_z))
    ld = [pltpu.async_copy(h_hbm.at[pl.ds(base + c * CH, CH)], hbuf.at[c],
                           sems[c]) for c in range(2)]
    _zero_rows(zbuf)
    zs = [pltpu.async_copy(zbuf, table.at[pl.ds(sid * (GTAB // NSUB) + z * CH, CH)],
                           sem_z) for z in range(GTAB // NSUB // CH)]
    for dsc in ins:
        dsc.wait()
    for dsc in zs:
        dsc.wait()
    ld2 = pltpu.async_copy(h_hbm.at[pl.ds(base + 2 * CH, CH)], zbuf, sem_z)
    for dsc in ld:
        dsc.wait()
    plsc.subcore_barrier()
    sc0 = pltpu.async_copy(hbuf.at[0], table.at[siv.at[0]], sem_a, add=True)
    sc1 = pltpu.async_copy(hbuf.at[1], table.at[siv.at[1]], sem_b, add=True)
    ld2.wait()
    sc2 = pltpu.async_copy(zbuf, table.at[siv.at[2]], sem_z, add=True)
    sc0.wait()
    pltpu.async_copy(h_hbm.at[pl.ds(base + 3 * CH, CH)], hbuf.at[0], sem_a).wait()
    sc3 = pltpu.async_copy(hbuf.at[0], table.at[siv.at[3]], sem_a, add=True)
    sc1.wait()
    sc2.wait()
    sc3.wait()
    plsc.subcore_barrier()
    g0 = pltpu.async_copy(table.at[giv.at[0]], hbuf.at[0], sem_a)
    g1 = pltpu.async_copy(table.at[giv.at[1]], hbuf.at[1], sem_b)
    g2 = pltpu.async_copy(table.at[giv.at[2]], zbuf, sem_z)
    g0.wait()
    o0 = pltpu.async_copy(hbuf.at[0], m_hbm.at[cid, pl.ds(base, CH)], sem_a)
    g1.wait()
    o1 = pltpu.async_copy(hbuf.at[1], m_hbm.at[cid, pl.ds(base + CH, CH)], sem_b)
    g2.wait()
    o2 = pltpu.async_copy(zbuf, m_hbm.at[cid, pl.ds(base + 2 * CH, CH)], sem_z)
    o0.wait()
    pltpu.async_copy(table.at[giv.at[3]], hbuf.at[0], sem_a).wait()
    o3 = pltpu.async_copy(hbuf.at[0], m_hbm.at[cid, pl.ds(base + 3 * CH, CH)],
                          sem_a)
    o1.wait()
    o2.wait()
    o3.wait()


@functools.partial(
    pl.kernel,
    out_type=jax.ShapeDtypeStruct((N, H), jnp.float32),
    mesh=_mesh,
    scratch_types=[
        pltpu.VMEM((NCHUNK, CH), jnp.int32),
        pltpu.VMEM((2, CH, H), jnp.float32),
        pltpu.VMEM((CH, H), jnp.float32),
        pltpu.VMEM_SHARED((NTAB, H), jnp.float32),
        pltpu.SemaphoreType.DMA,
        pltpu.SemaphoreType.DMA,
        pltpu.SemaphoreType.DMA,
    ],
)
def _sc_final(hw_hbm, dstp_hbm, xa_hbm, out_hbm, iv, hbuf, zbuf, table,
              sem_a, sem_b, sem_z):
    cid = lax.axis_index("c")
    sid = lax.axis_index("s")
    base = sid * EPT
    sems = (sem_a, sem_b)
    ins = [pltpu.async_copy(dstp_hbm.at[pl.ds(base + c * CH, CH)], iv.at[c],
                            sem_z) for c in range(NCHUNK)]
    ld = [pltpu.async_copy(hw_hbm.at[pl.ds(base + c * CH, CH)], hbuf.at[c],
                           sems[c]) for c in range(2)]
    _zero_rows(zbuf)
    rpt = NTAB // NSUB
    z = pltpu.async_copy(zbuf.at[pl.ds(0, rpt)], table.at[pl.ds(sid * rpt, rpt)],
                         sem_z)
    for dsc in ins:
        dsc.wait()
    z.wait()
    ld2 = pltpu.async_copy(hw_hbm.at[pl.ds(base + 2 * CH, CH)], zbuf, sem_z)
    for dsc in ld:
        dsc.wait()
    plsc.subcore_barrier()
    sc0 = pltpu.async_copy(hbuf.at[0], table.at[iv.at[0]], sem_a, add=True)
    sc1 = pltpu.async_copy(hbuf.at[1], table.at[iv.at[1]], sem_b, add=True)
    ld2.wait()
    sc2 = pltpu.async_copy(zbuf, table.at[iv.at[2]], sem_z, add=True)
    sc0.wait()
    pltpu.async_copy(hw_hbm.at[pl.ds(base + 3 * CH, CH)], hbuf.at[0], sem_a).wait()
    sc3 = pltpu.async_copy(hbuf.at[0], table.at[iv.at[3]], sem_a, add=True)
    sc1.wait()
    sc2.wait()
    sc3.wait()
    plsc.subcore_barrier()
    rpo = 512 // NSUB
    row0 = cid * 512 + sid * rpo
    pltpu.sync_copy(xa_hbm.at[pl.ds(row0, rpo)], hbuf.at[0, pl.ds(0, rpo)])
    pltpu.sync_copy(table.at[pl.ds(row0, rpo)], hbuf.at[1, pl.ds(0, rpo)])

    def orow(r, _):
        for c in range(H // 16):
            sl = pl.ds(c * 16, 16)
            hbuf[0, r, sl] = jnp.maximum(hbuf[0, r, sl] + hbuf[1, r, sl], 0.0)
        return 0

    lax.fori_loop(0, rpo, orow, 0)
    last = jnp.logical_and(cid == 1, sid == NSUB - 1)

    @pl.when(jnp.logical_not(last))
    def _():
        pltpu.sync_copy(hbuf.at[0, pl.ds(0, rpo)], out_hbm.at[pl.ds(row0, rpo)])

    @pl.when(last)
    def _():
        pltpu.sync_copy(hbuf.at[0, pl.ds(0, N - 992)],
                        out_hbm.at[pl.ds(992, N - 992)])


def _eb_body(ea_ref, w2_ref, b_ref, o_ref):
    o_ref[...] = (jnp.dot(ea_ref[...], w2_ref[...],
                          preferred_element_type=jnp.float32) + b_ref[...])


def _tc_eb(ea, w2t, b):
    return pl.pallas_call(
        _eb_body,
        grid=(EP // 1024,),
        in_specs=[
            pl.BlockSpec((1024, DE), lambda i: (i, 0)),
            pl.BlockSpec((DE, H), lambda i: (0, 0)),
            pl.BlockSpec((1, H), lambda i: (0, 0)),
        ],
        out_specs=pl.BlockSpec((1024, H), lambda i: (i, 0)),
        out_shape=jax.ShapeDtypeStruct((EP, H), jnp.float32),
    )(ea, w2t, b)


def _round_body(m_ref, h0_ref, w_ref, b_ref, o_ref):
    mm = m_ref[0] - m_ref[1]
    acc = (h0_ref[...]
           + jnp.dot(mm, w_ref[...], preferred_element_type=jnp.float32)
           + b_ref[...])
    o_ref[...] = jnp.maximum(acc, 0.0)


def _tc_round(m, h0, wt, b):
    return pl.pallas_call(
        _round_body,
        grid=(EP // 2048,),
        in_specs=[
            pl.BlockSpec((2, 2048, H), lambda i: (0, i, 0)),
            pl.BlockSpec((2048, H), lambda i: (i, 0)),
            pl.BlockSpec((H, H), lambda i: (0, 0)),
            pl.BlockSpec((1, H), lambda i: (0, 0)),
        ],
        out_specs=pl.BlockSpec((2048, H), lambda i: (i, 0)),
        out_shape=jax.ShapeDtypeStruct((EP, H), jnp.float32),
    )(m, h0, wt, b)


def _round_final_body(m_ref, h0_ref, w_ref, b_ref, w2_ref, o_ref):
    mm = m_ref[0] - m_ref[1]
    h3 = jnp.maximum(
        h0_ref[...]
        + jnp.dot(mm, w_ref[...], preferred_element_type=jnp.float32)
        + b_ref[...], 0.0)
    o_ref[...] = jnp.dot(h3, w2_ref[...], preferred_element_type=jnp.float32)


def _tc_round_final(m, h0, wt, b, wn2t):
    return pl.pallas_call(
        _round_final_body,
        grid=(EP // 2048,),
        in_specs=[
            pl.BlockSpec((2, 2048, H), lambda i: (0, i, 0)),
            pl.BlockSpec((2048, H), lambda i: (i, 0)),
            pl.BlockSpec((H, H), lambda i: (0, 0)),
            pl.BlockSpec((1, H), lambda i: (0, 0)),
            pl.BlockSpec((H, H), lambda i: (0, 0)),
        ],
        out_specs=pl.BlockSpec((2048, H), lambda i: (i, 0)),
        out_shape=jax.ShapeDtypeStruct((EP, H), jnp.float32),
    )(m, h0, wt, b, wn2t)


def _pre_body(x_ref, w1_ref, wn_ref, b_ref, xw_ref, xa_ref):
    xw_ref[...] = jnp.dot(x_ref[...], w1_ref[...],
                          preferred_element_type=jnp.float32)
    xa_ref[...] = (jnp.dot(x_ref[...], wn_ref[...],
                           preferred_element_type=jnp.float32) + b_ref[...])


def _tc_pre(xp, w1t, wn1t, b):
    return pl.pallas_call(
        _pre_body,
        grid=(NTAB // 512,),
        in_specs=[
            pl.BlockSpec((512, D), lambda i: (i, 0)),
            pl.BlockSpec((D, H), lambda i: (0, 0)),
            pl.BlockSpec((D, H), lambda i: (0, 0)),
            pl.BlockSpec((1, H), lambda i: (0, 0)),
        ],
        out_specs=[
            pl.BlockSpec((512, H), lambda i: (i, 0)),
            pl.BlockSpec((512, H), lambda i: (i, 0)),
        ],
        out_shape=[
            jax.ShapeDtypeStruct((NTAB, H), jnp.float32),
            jax.ShapeDtypeStruct((NTAB, H), jnp.float32),
        ],
    )(xp, w1t, wn1t, b)


def kernel(x, edge_index, edge_attr, W_edge_init, b_edge_init, W_msg, b_msg,
           W_node, b_node):
    src = edge_index[0]
    dst = edge_index[1]

    pad = EP - E
    srcp = jnp.concatenate([src, jnp.full((pad,), N, jnp.int32)])
    dstp = jnp.concatenate([dst, jnp.full((pad,), N, jnp.int32)])

    w1t = W_edge_init[:, :D].T
    w2t = W_edge_init[:, D:].T
    wmt = W_msg.T
    wn1t = W_node[:, :D].T
    wn2t = W_node[:, D:].T
    be = b_edge_init.reshape(1, H)
    bm = b_msg.reshape(1, H)
    bn = b_node.reshape(1, H)

    xw, xa = _tc_pre(x, w1t, wn1t, bn)
    eb = _tc_eb(edge_attr, w2t, be)
    sidx, gidx, h0 = _sc_prep(xw, eb, srcp, dstp)
    h = h0
    for t in range(T - 1):
        m = _sc_round(h, sidx, gidx)
        h = _tc_round(m, h0, wmt, bm)
    m = _sc_round(h, sidx, gidx)
    hw = _tc_round_final(m, h0, wmt, bm, wn2t)
    return _sc_final(hw, dstp, xa)
